# baseline (speedup 1.0000x reference)
import functools

import jax
import jax.numpy as jnp
from jax import lax
from jax.experimental import pallas as pl
from jax.experimental.pallas import tpu as pltpu
import jax.experimental.pallas.tpu_sc as plsc

N = 10000
NP = 10240
E = 160000
EP = 163840
D_IN = 256
NWID = 512
ENC_NH = 512
NT = 128
NDOC = 64
EPS = 1e-5
NC = 2
NS = 16
EDGE_K = 80
BLK = 1024
GRID = NP // BLK

def _sc_mesh():
    return plsc.VectorSubcoreMesh(
        core_axis_name="c", subcore_axis_name="s",
        num_cores=NC, num_subcores=NS)


def _make_sc_gather(V, D, B, K):
    bpw = B // (NC * NS)
    nchunks = bpw // K

    def body(table_hbm, idx_hbm, out_hbm, idx_v, rows_v, sem):
        wid = lax.axis_index("s") * NC + lax.axis_index("c")
        base = wid * bpw

        def chunk(j, carry):
            off = base + j * K
            pltpu.sync_copy(idx_hbm.at[pl.ds(off, K)], idx_v)
            pltpu.async_copy(table_hbm.at[idx_v], rows_v, sem).wait()
            pltpu.sync_copy(rows_v, out_hbm.at[pl.ds(off, K)])
            return carry

        lax.fori_loop(0, nchunks, chunk, 0)

    return pl.kernel(
        body,
        out_type=jax.ShapeDtypeStruct((B, D), jnp.float32),
        scratch_types=[
            pltpu.VMEM((K,), jnp.int32),
            pltpu.VMEM((K, D), jnp.float32),
            pltpu.SemaphoreType.DMA,
        ],
        mesh=_sc_mesh(),
    )


def _make_sc_edge(K=EDGE_K):
    nchunks_total = EP // K
    ntile = nchunks_total // NS
    zrows = NP // NS

    def body(xv_hbm, epk_hbm, ewr_hbm, zeros_hbm, out_hbm,
             ebuf, ewb, rows, dstb, acc,
             isem0, isem1, gsem0, gsem1, ssem0, ssem1):
        c = lax.axis_index("c")
        s = lax.axis_index("s")
        isem = (isem0, isem1)
        gsem = (gsem0, gsem1)
        ssem = (ssem0, ssem1)
        pltpu.sync_copy(zeros_hbm.at[pl.ds(s * zrows, zrows)],
                        acc.at[pl.ds(s * zrows, zrows)])
        plsc.subcore_barrier()
        base = s * ntile

        def issue_idx(cid, sl):
            pltpu.async_copy(epk_hbm.at[cid], ebuf.at[sl], isem[sl])
            pltpu.async_copy(ewr_hbm.at[cid], ewb.at[sl], isem[sl])

        def wait_idx(sl):
            pltpu.make_async_copy(epk_hbm.at[0], ebuf.at[sl], isem[sl]).wait()
            pltpu.make_async_copy(ewr_hbm.at[0], ewb.at[sl], isem[sl]).wait()

        def issue_gather(sl):
            pltpu.async_copy(xv_hbm.at[ebuf.at[sl, 0], pl.ds(c, 1), :],
                             rows.at[sl], gsem[sl])

        def wait_gather(sl):
            pltpu.make_async_copy(xv_hbm.at[ebuf.at[sl, 0], pl.ds(c, 1), :],
                                  rows.at[sl], gsem[sl]).wait()

        def issue_scatter(sl):
            pltpu.async_copy(rows.at[sl], acc.at[dstb.at[sl]], ssem[sl],
                             add=True)

        def wait_scatter(sl):
            pltpu.make_async_copy(rows.at[sl], acc.at[dstb.at[sl]],
                                  ssem[sl]).wait()

        def scale(sl):
            @plsc.parallel_loop(0, K, step=1, unroll=4)
            def _(e):
                ew_bc = ewb[sl, e, pl.ds(0, 16)]
                for jj in range(8):
                    cs = pl.ds(jj * 16, 16)
                    rows[sl, e, 0, cs] = rows[sl, e, 0, cs] * ew_bc

        def step(j, sl, first=False):
            o = 1 - sl
            if not first:
                wait_scatter(o)
            wait_idx(o)
            issue_gather(o)
            wait_gather(sl)
            scale(sl)
            for q in range(K // 16):
                qs = pl.ds(q * 16, 16)
                dstb[sl, qs] = ebuf[sl, 1, qs]
            issue_idx(base + j + 2, sl)
            issue_scatter(sl)

        issue_idx(base, 0)
        issue_idx(base + 1, 1)
        wait_idx(0)
        issue_gather(0)
        step(0, 0, first=True)

        def pair(jj, carry):
            j0 = 1 + 2 * jj
            step(j0, 1)
            step(j0 + 1, 0)
            return carry

        lax.fori_loop(0, (ntile - 2) // 2, pair, 0)
        step(ntile - 1, 1)
        wait_idx(1)
        wait_gather(0)
        wait_scatter(1)
        plsc.subcore_barrier()
        pltpu.sync_copy(acc.at[pl.ds(s * zrows, zrows)],
                        out_hbm.at[pl.ds(s * zrows, zrows), pl.ds(c, 1), :])

    return pl.kernel(
        body,
        out_type=jax.ShapeDtypeStruct((NP, 2, 128), jnp.float32),
        scratch_types=[
            pltpu.VMEM((2, 2, K), jnp.int32),
            pltpu.VMEM((2, K, 16), jnp.float32),
            pltpu.VMEM((2, K, 1, 128), jnp.float32),
            pltpu.VMEM((2, K), jnp.int32),
            pltpu.VMEM_SHARED((NP, 1, 128), jnp.float32),
            pltpu.SemaphoreType.DMA,
            pltpu.SemaphoreType.DMA,
            pltpu.SemaphoreType.DMA,
            pltpu.SemaphoreType.DMA,
            pltpu.SemaphoreType.DMA,
            pltpu.SemaphoreType.DMA,
        ],
        mesh=_sc_mesh(),
    )


def _bdot(a, b):
    return jnp.dot(a.astype(jnp.bfloat16), b.astype(jnp.bfloat16),
                   preferred_element_type=jnp.float32)


def _h_body(agg_ref, x_ref, iw_ref, wrel_ref, wroot_ref, brel_ref,
            h_ref, st_ref):
    i = pl.program_id(0)
    x = x_ref[...]
    agg = agg_ref[...] + iw_ref[...] * x
    h = _bdot(agg, wrel_ref[...]) + _bdot(x, wroot_ref[...]) + brel_ref[...]
    h_ref[...] = h
    gid = i * BLK + lax.broadcasted_iota(jnp.int32, (BLK, 1), 0)
    mf = (gid < N).astype(jnp.float32)
    hm = h * mf
    s0 = jnp.sum(hm, axis=0, keepdims=True)
    s1 = jnp.sum(h * hm, axis=0, keepdims=True)
    blkstats = jnp.concatenate(
        [s0, s1, jnp.zeros((6, NWID), jnp.float32)], axis=0)

    @pl.when(i == 0)
    def _():
        st_ref[...] = jnp.zeros_like(st_ref)

    st_ref[...] += blkstats


def _tc_h():
    return pl.pallas_call(
        _h_body,
        grid=(GRID,),
        in_specs=[
            pl.BlockSpec((BLK, D_IN), lambda i: (i, 0)),
            pl.BlockSpec((BLK, D_IN), lambda i: (i, 0)),
            pl.BlockSpec((BLK, 1), lambda i: (i, 0)),
            pl.BlockSpec((D_IN, NWID), lambda i: (0, 0)),
            pl.BlockSpec((D_IN, NWID), lambda i: (0, 0)),
            pl.BlockSpec((1, NWID), lambda i: (0, 0)),
        ],
        out_specs=[
            pl.BlockSpec((BLK, NWID), lambda i: (i, 0)),
            pl.BlockSpec((8, NWID), lambda i: (0, 0)),
        ],
        out_shape=[
            jax.ShapeDtypeStruct((NP, NWID), jnp.float32),
            jax.ShapeDtypeStruct((8, NWID), jnp.float32),
        ],
    )


def _enc_body(h_ref, st_ref, x_ref, xb_ref, g1_ref, b1_ref,
              wf1_ref, bf1_ref, wf2_ref, bf2_ref,
              enc1_ref, doc_ref):
    i = pl.program_id(0)
    m = st_ref[0, :] * (1.0 / N)
    v = st_ref[1, :] * (1.0 / N) - m * m
    inv = lax.rsqrt(v + EPS)
    t = jnp.tanh((h_ref[...] - m[None, :]) * inv[None, :] * g1_ref[...]
                 + b1_ref[...])
    enc1 = jnp.concatenate([t, x_ref[...]], axis=1)
    enc1_ref[...] = enc1
    a = jax.nn.sigmoid(_bdot(enc1, wf1_ref[...]) + bf1_ref[...])
    b = jnp.tanh(_bdot(enc1, wf2_ref[...]) + bf2_ref[...])
    enc2 = a * b
    oh = (xb_ref[...] ==
          lax.broadcasted_iota(jnp.int32, (1, NDOC), 1)).astype(jnp.float32)
    part = lax.dot_general(oh, enc2, (((0,), (0,)), ((), ())),
                           preferred_element_type=jnp.float32, precision=lax.Precision.HIGHEST)

    @pl.when(i == 0)
    def _():
        doc_ref[...] = jnp.zeros_like(doc_ref)

    doc_ref[...] += part


def _tc_enc():
    return pl.pallas_call(
        _enc_body,
        grid=(GRID,),
        in_specs=[
            pl.BlockSpec((BLK, NWID), lambda i: (i, 0)),
            pl.BlockSpec((8, NWID), lambda i: (0, 0)),
            pl.BlockSpec((BLK, D_IN), lambda i: (i, 0)),
            pl.BlockSpec((BLK, 1), lambda i: (i, 0)),
            pl.BlockSpec((1, NWID), lambda i: (0, 0)),
            pl.BlockSpec((1, NWID), lambda i: (0, 0)),
            pl.BlockSpec((NWID + D_IN, ENC_NH), lambda i: (0, 0)),
            pl.BlockSpec((1, ENC_NH), lambda i: (0, 0)),
            pl.BlockSpec((NWID + D_IN, ENC_NH), lambda i: (0, 0)),
            pl.BlockSpec((1, ENC_NH), lambda i: (0, 0)),
        ],
        out_specs=[
            pl.BlockSpec((BLK, NWID + D_IN), lambda i: (i, 0)),
            pl.BlockSpec((NDOC, ENC_NH), lambda i: (0, 0)),
        ],
        out_shape=[
            jax.ShapeDtypeStruct((NP, NWID + D_IN), jnp.float32),
            jax.ShapeDtypeStruct((NDOC, ENC_NH), jnp.float32),
        ],
    )


def _doc_body(d_ref, wm_ref, bm_ref, gm_ref, btm_ref, wl_ref, bl_ref,
              wpb_ref, mean_ref, logvar_ref, dp_ref):
    d = d_ref[...]
    mp = _bdot(d, wm_ref[...]) + bm_ref[...]
    mm = jnp.mean(mp, axis=0, keepdims=True)
    vv = jnp.mean(mp * mp, axis=0, keepdims=True) - mm * mm
    mean_ref[...] = (mp - mm) * lax.rsqrt(vv + EPS) * gm_ref[...] + btm_ref[...]
    logvar_ref[...] = _bdot(d, wl_ref[...]) + bl_ref[...]
    dp_ref[...] = _bdot(d, wpb_ref[...])


def _tc_doc():
    return pl.pallas_call(
        _doc_body,
        out_shape=[
            jax.ShapeDtypeStruct((NDOC, NT), jnp.float32),
            jax.ShapeDtypeStruct((NDOC, NT), jnp.float32),
            jax.ShapeDtypeStruct((NDOC, NT), jnp.float32),
        ],
    )


def _phi_body(enc1_ref, xb_ref, dp_ref, wpa_ref, bp_ref, phi_ref):
    oh = (xb_ref[...] ==
          lax.broadcasted_iota(jnp.int32, (1, NDOC), 1)).astype(jnp.float32)
    logits = (_bdot(enc1_ref[...], wpa_ref[...])
              + jnp.dot(oh, dp_ref[...], preferred_element_type=jnp.float32,
                        precision=lax.Precision.HIGHEST)
              + bp_ref[...])
    z = logits - jnp.max(logits, axis=1, keepdims=True)
    ez = jnp.exp(z)
    phi_ref[...] = ez / jnp.sum(ez, axis=1, keepdims=True)


def _tc_phi():
    return pl.pallas_call(
        _phi_body,
        grid=(GRID,),
        in_specs=[
            pl.BlockSpec((BLK, NWID + D_IN), lambda i: (i, 0)),
            pl.BlockSpec((BLK, 1), lambda i: (i, 0)),
            pl.BlockSpec((NDOC, NT), lambda i: (0, 0)),
            pl.BlockSpec((NWID + D_IN, NT), lambda i: (0, 0)),
            pl.BlockSpec((1, NT), lambda i: (0, 0)),
        ],
        out_specs=pl.BlockSpec((BLK, NT), lambda i: (i, 0)),
        out_shape=jax.ShapeDtypeStruct((NP, NT), jnp.float32),
    )


def kernel(idx_x, idx_w, x_batch, edge_index, edge_weight, word_vec,
           W_rel, b_rel, W_root, bn1_gamma, bn1_beta,
           W_fc1, b_fc1, W_fc2, b_fc2,
           W_mean, b_mean, bn_mean_gamma, bn_mean_beta,
           W_logvar, b_logvar, W_phi, b_phi):
    f32 = jnp.float32
    idx_pad = jnp.pad(idx_x.astype(jnp.int32), (0, NP - N))
    src = jnp.pad(edge_index[0].astype(jnp.int32), (0, EP - E))
    dst = jnp.pad(edge_index[1].astype(jnp.int32), (0, EP - E))
    ew_pad = jnp.pad(edge_weight, (0, EP - E))
    epk = jnp.pad(jnp.stack([src.reshape(-1, EDGE_K), dst.reshape(-1, EDGE_K)],
                            axis=1),
                  ((0, 2), (0, 0), (0, 0)))
    ew_rep = jnp.pad(
        jnp.broadcast_to(ew_pad.reshape(EP // EDGE_K, EDGE_K, 1),
                         (EP // EDGE_K, EDGE_K, 16)),
        ((0, 2), (0, 0), (0, 0)))
    iw = jnp.pad(idx_w, (0, NP - N)).reshape(NP, 1)
    xb = jnp.pad(x_batch, (0, NP - N), constant_values=NDOC).reshape(NP, 1)
    zeros_half = jnp.zeros((NP, 1, 128), f32)

    x = _make_sc_gather(50000, D_IN, NP, 320)(word_vec, idx_pad)
    agg = _make_sc_edge()(x.reshape(NP, 2, 128), epk, ew_rep,
                          zeros_half).reshape(NP, D_IN)

    h, stats = _tc_h()(
        agg, x, iw, W_rel, W_root, b_rel.reshape(1, NWID))
    enc1, doc = _tc_enc()(
        h, stats, x, xb, bn1_gamma.reshape(1, NWID), bn1_beta.reshape(1, NWID),
        W_fc1, b_fc1.reshape(1, ENC_NH), W_fc2, b_fc2.reshape(1, ENC_NH))
    mean, logvar, docproj = _tc_doc()(
        doc, W_mean, b_mean.reshape(1, NT), bn_mean_gamma.reshape(1, NT),
        bn_mean_beta.reshape(1, NT), W_logvar, b_logvar.reshape(1, NT),
        W_phi[NWID + D_IN:, :])
    phi = _tc_phi()(enc1, xb, docproj, W_phi[:NWID + D_IN, :],
                    b_phi.reshape(1, NT))
    return (mean, logvar, phi[:N])


# speedup vs baseline: 1.0008x; 1.0008x over previous
import functools

import jax
import jax.numpy as jnp
from jax import lax
from jax.experimental import pallas as pl
from jax.experimental.pallas import tpu as pltpu
import jax.experimental.pallas.tpu_sc as plsc

N = 10000
NP = 10240
E = 160000
EP = 163840
D_IN = 256
NWID = 512
ENC_NH = 512
NT = 128
NDOC = 64
EPS = 1e-5
NC = 2
NS = 16
EDGE_K = 80
BLK = 1024
GRID = NP // BLK

def _sc_mesh():
    return plsc.VectorSubcoreMesh(
        core_axis_name="c", subcore_axis_name="s",
        num_cores=NC, num_subcores=NS)


def _make_sc_gather(V, D, B, K):
    bpw = B // (NC * NS)
    nchunks = bpw // K

    def body(table_hbm, idx_hbm, out_hbm, idx_v, rows_v, sem):
        wid = lax.axis_index("s") * NC + lax.axis_index("c")
        base = wid * bpw

        def chunk(j, carry):
            off = base + j * K
            pltpu.sync_copy(idx_hbm.at[pl.ds(off, K)], idx_v)
            pltpu.async_copy(table_hbm.at[idx_v], rows_v, sem).wait()
            pltpu.sync_copy(rows_v, out_hbm.at[pl.ds(off, K)])
            return carry

        lax.fori_loop(0, nchunks, chunk, 0)

    return pl.kernel(
        body,
        out_type=jax.ShapeDtypeStruct((B, D), jnp.float32),
        scratch_types=[
            pltpu.VMEM((K,), jnp.int32),
            pltpu.VMEM((K, D), jnp.float32),
            pltpu.SemaphoreType.DMA,
        ],
        mesh=_sc_mesh(),
    )


def _make_sc_edge(K=EDGE_K):
    nchunks_total = EP // K
    ntile = nchunks_total // NS
    zrows = NP // NS

    def body(xv_hbm, epk_hbm, ewr_hbm, zeros_hbm, out_hbm,
             ebuf, ewb, rows, dstb, acc,
             isem0, isem1, gsem0, gsem1, ssem0, ssem1):
        c = lax.axis_index("c")
        s = lax.axis_index("s")
        isem = (isem0, isem1)
        gsem = (gsem0, gsem1)
        ssem = (ssem0, ssem1)
        pltpu.sync_copy(zeros_hbm.at[pl.ds(s * zrows, zrows)],
                        acc.at[pl.ds(s * zrows, zrows)])
        plsc.subcore_barrier()
        base = s * ntile

        def issue_idx(cid, sl):
            pltpu.async_copy(epk_hbm.at[cid], ebuf.at[sl], isem[sl])
            pltpu.async_copy(ewr_hbm.at[cid], ewb.at[sl], isem[sl])

        def wait_idx(sl):
            pltpu.make_async_copy(epk_hbm.at[0], ebuf.at[sl], isem[sl]).wait()
            pltpu.make_async_copy(ewr_hbm.at[0], ewb.at[sl], isem[sl]).wait()

        H = K // 2

        def issue_gather(sl):
            pltpu.async_copy(xv_hbm.at[ebuf.at[sl, 0, pl.ds(0, H)],
                                       pl.ds(c, 1), :],
                             rows.at[sl, pl.ds(0, H)], gsem[sl])
            pltpu.async_copy(xv_hbm.at[ebuf.at[sl, 0, pl.ds(H, H)],
                                       pl.ds(c, 1), :],
                             rows.at[sl, pl.ds(H, H)], gsem[sl])

        def wait_gather(sl):
            pltpu.make_async_copy(xv_hbm.at[ebuf.at[sl, 0, pl.ds(0, H)],
                                            pl.ds(c, 1), :],
                                  rows.at[sl, pl.ds(0, H)], gsem[sl]).wait()
            pltpu.make_async_copy(xv_hbm.at[ebuf.at[sl, 0, pl.ds(H, H)],
                                            pl.ds(c, 1), :],
                                  rows.at[sl, pl.ds(H, H)], gsem[sl]).wait()

        def issue_scatter(sl):
            pltpu.async_copy(rows.at[sl], acc.at[dstb.at[sl]], ssem[sl],
                             add=True)

        def wait_scatter(sl):
            pltpu.make_async_copy(rows.at[sl], acc.at[dstb.at[sl]],
                                  ssem[sl]).wait()

        def scale(sl):
            @plsc.parallel_loop(0, K, step=1, unroll=4)
            def _(e):
                ew_bc = ewb[sl, e, pl.ds(0, 16)]
                for jj in range(8):
                    cs = pl.ds(jj * 16, 16)
                    rows[sl, e, 0, cs] = rows[sl, e, 0, cs] * ew_bc

        def step(j, sl, first=False):
            o = 1 - sl
            if not first:
                wait_scatter(o)
            wait_idx(o)
            issue_gather(o)
            wait_gather(sl)
            scale(sl)
            for q in range(K // 16):
                qs = pl.ds(q * 16, 16)
                dstb[sl, qs] = ebuf[sl, 1, qs]
            issue_idx(base + j + 2, sl)
            issue_scatter(sl)

        issue_idx(base, 0)
        issue_idx(base + 1, 1)
        wait_idx(0)
        issue_gather(0)
        step(0, 0, first=True)

        def pair(jj, carry):
            j0 = 1 + 2 * jj
            step(j0, 1)
            step(j0 + 1, 0)
            return carry

        lax.fori_loop(0, (ntile - 2) // 2, pair, 0)
        step(ntile - 1, 1)
        wait_idx(1)
        wait_gather(0)
        wait_scatter(1)
        plsc.subcore_barrier()
        pltpu.sync_copy(acc.at[pl.ds(s * zrows, zrows)],
                        out_hbm.at[pl.ds(s * zrows, zrows), pl.ds(c, 1), :])

    return pl.kernel(
        body,
        out_type=jax.ShapeDtypeStruct((NP, 2, 128), jnp.float32),
        scratch_types=[
            pltpu.VMEM((2, 2, K), jnp.int32),
            pltpu.VMEM((2, K, 16), jnp.float32),
            pltpu.VMEM((2, K, 1, 128), jnp.float32),
            pltpu.VMEM((2, K), jnp.int32),
            pltpu.VMEM_SHARED((NP, 1, 128), jnp.float32),
            pltpu.SemaphoreType.DMA,
            pltpu.SemaphoreType.DMA,
            pltpu.SemaphoreType.DMA,
            pltpu.SemaphoreType.DMA,
            pltpu.SemaphoreType.DMA,
            pltpu.SemaphoreType.DMA,
        ],
        mesh=_sc_mesh(),
    )


def _bdot(a, b):
    return jnp.dot(a.astype(jnp.bfloat16), b.astype(jnp.bfloat16),
                   preferred_element_type=jnp.float32)


def _h_body(agg_ref, x_ref, iw_ref, wrel_ref, wroot_ref, brel_ref,
            h_ref, st_ref):
    i = pl.program_id(0)
    x = x_ref[...]
    agg = agg_ref[...] + iw_ref[...] * x
    h = _bdot(agg, wrel_ref[...]) + _bdot(x, wroot_ref[...]) + brel_ref[...]
    h_ref[...] = h
    gid = i * BLK + lax.broadcasted_iota(jnp.int32, (BLK, 1), 0)
    mf = (gid < N).astype(jnp.float32)
    hm = h * mf
    s0 = jnp.sum(hm, axis=0, keepdims=True)
    s1 = jnp.sum(h * hm, axis=0, keepdims=True)
    blkstats = jnp.concatenate(
        [s0, s1, jnp.zeros((6, NWID), jnp.float32)], axis=0)

    @pl.when(i == 0)
    def _():
        st_ref[...] = jnp.zeros_like(st_ref)

    st_ref[...] += blkstats


def _tc_h():
    return pl.pallas_call(
        _h_body,
        grid=(GRID,),
        in_specs=[
            pl.BlockSpec((BLK, D_IN), lambda i: (i, 0)),
            pl.BlockSpec((BLK, D_IN), lambda i: (i, 0)),
            pl.BlockSpec((BLK, 1), lambda i: (i, 0)),
            pl.BlockSpec((D_IN, NWID), lambda i: (0, 0)),
            pl.BlockSpec((D_IN, NWID), lambda i: (0, 0)),
            pl.BlockSpec((1, NWID), lambda i: (0, 0)),
        ],
        out_specs=[
            pl.BlockSpec((BLK, NWID), lambda i: (i, 0)),
            pl.BlockSpec((8, NWID), lambda i: (0, 0)),
        ],
        out_shape=[
            jax.ShapeDtypeStruct((NP, NWID), jnp.float32),
            jax.ShapeDtypeStruct((8, NWID), jnp.float32),
        ],
    )


def _enc_body(h_ref, st_ref, x_ref, xb_ref, g1_ref, b1_ref,
              wf1_ref, bf1_ref, wf2_ref, bf2_ref,
              enc1_ref, doc_ref):
    i = pl.program_id(0)
    m = st_ref[0, :] * (1.0 / N)
    v = st_ref[1, :] * (1.0 / N) - m * m
    inv = lax.rsqrt(v + EPS)
    t = jnp.tanh((h_ref[...] - m[None, :]) * inv[None, :] * g1_ref[...]
                 + b1_ref[...])
    enc1 = jnp.concatenate([t, x_ref[...]], axis=1)
    enc1_ref[...] = enc1
    a = jax.nn.sigmoid(_bdot(enc1, wf1_ref[...]) + bf1_ref[...])
    b = jnp.tanh(_bdot(enc1, wf2_ref[...]) + bf2_ref[...])
    enc2 = a * b
    oh = (xb_ref[...] ==
          lax.broadcasted_iota(jnp.int32, (1, NDOC), 1)).astype(jnp.float32)
    part = lax.dot_general(oh, enc2, (((0,), (0,)), ((), ())),
                           preferred_element_type=jnp.float32, precision=lax.Precision.HIGHEST)

    @pl.when(i == 0)
    def _():
        doc_ref[...] = jnp.zeros_like(doc_ref)

    doc_ref[...] += part


def _tc_enc():
    return pl.pallas_call(
        _enc_body,
        grid=(GRID,),
        in_specs=[
            pl.BlockSpec((BLK, NWID), lambda i: (i, 0)),
            pl.BlockSpec((8, NWID), lambda i: (0, 0)),
            pl.BlockSpec((BLK, D_IN), lambda i: (i, 0)),
            pl.BlockSpec((BLK, 1), lambda i: (i, 0)),
            pl.BlockSpec((1, NWID), lambda i: (0, 0)),
            pl.BlockSpec((1, NWID), lambda i: (0, 0)),
            pl.BlockSpec((NWID + D_IN, ENC_NH), lambda i: (0, 0)),
            pl.BlockSpec((1, ENC_NH), lambda i: (0, 0)),
            pl.BlockSpec((NWID + D_IN, ENC_NH), lambda i: (0, 0)),
            pl.BlockSpec((1, ENC_NH), lambda i: (0, 0)),
        ],
        out_specs=[
            pl.BlockSpec((BLK, NWID + D_IN), lambda i: (i, 0)),
            pl.BlockSpec((NDOC, ENC_NH), lambda i: (0, 0)),
        ],
        out_shape=[
            jax.ShapeDtypeStruct((NP, NWID + D_IN), jnp.float32),
            jax.ShapeDtypeStruct((NDOC, ENC_NH), jnp.float32),
        ],
    )


def _doc_body(d_ref, wm_ref, bm_ref, gm_ref, btm_ref, wl_ref, bl_ref,
              wpb_ref, mean_ref, logvar_ref, dp_ref):
    d = d_ref[...]
    mp = _bdot(d, wm_ref[...]) + bm_ref[...]
    mm = jnp.mean(mp, axis=0, keepdims=True)
    vv = jnp.mean(mp * mp, axis=0, keepdims=True) - mm * mm
    mean_ref[...] = (mp - mm) * lax.rsqrt(vv + EPS) * gm_ref[...] + btm_ref[...]
    logvar_ref[...] = _bdot(d, wl_ref[...]) + bl_ref[...]
    dp_ref[...] = _bdot(d, wpb_ref[...])


def _tc_doc():
    return pl.pallas_call(
        _doc_body,
        out_shape=[
            jax.ShapeDtypeStruct((NDOC, NT), jnp.float32),
            jax.ShapeDtypeStruct((NDOC, NT), jnp.float32),
            jax.ShapeDtypeStruct((NDOC, NT), jnp.float32),
        ],
    )


def _phi_body(enc1_ref, xb_ref, dp_ref, wpa_ref, bp_ref, phi_ref):
    oh = (xb_ref[...] ==
          lax.broadcasted_iota(jnp.int32, (1, NDOC), 1)).astype(jnp.float32)
    logits = (_bdot(enc1_ref[...], wpa_ref[...])
              + jnp.dot(oh, dp_ref[...], preferred_element_type=jnp.float32,
                        precision=lax.Precision.HIGHEST)
              + bp_ref[...])
    z = logits - jnp.max(logits, axis=1, keepdims=True)
    ez = jnp.exp(z)
    phi_ref[...] = ez / jnp.sum(ez, axis=1, keepdims=True)


def _tc_phi():
    return pl.pallas_call(
        _phi_body,
        grid=(GRID,),
        in_specs=[
            pl.BlockSpec((BLK, NWID + D_IN), lambda i: (i, 0)),
            pl.BlockSpec((BLK, 1), lambda i: (i, 0)),
            pl.BlockSpec((NDOC, NT), lambda i: (0, 0)),
            pl.BlockSpec((NWID + D_IN, NT), lambda i: (0, 0)),
            pl.BlockSpec((1, NT), lambda i: (0, 0)),
        ],
        out_specs=pl.BlockSpec((BLK, NT), lambda i: (i, 0)),
        out_shape=jax.ShapeDtypeStruct((NP, NT), jnp.float32),
    )


def kernel(idx_x, idx_w, x_batch, edge_index, edge_weight, word_vec,
           W_rel, b_rel, W_root, bn1_gamma, bn1_beta,
           W_fc1, b_fc1, W_fc2, b_fc2,
           W_mean, b_mean, bn_mean_gamma, bn_mean_beta,
           W_logvar, b_logvar, W_phi, b_phi):
    f32 = jnp.float32
    idx_pad = jnp.pad(idx_x.astype(jnp.int32), (0, NP - N))
    src = jnp.pad(edge_index[0].astype(jnp.int32), (0, EP - E))
    dst = jnp.pad(edge_index[1].astype(jnp.int32), (0, EP - E))
    ew_pad = jnp.pad(edge_weight, (0, EP - E))
    epk = jnp.pad(jnp.stack([src.reshape(-1, EDGE_K), dst.reshape(-1, EDGE_K)],
                            axis=1),
                  ((0, 2), (0, 0), (0, 0)))
    ew_rep = jnp.pad(
        jnp.broadcast_to(ew_pad.reshape(EP // EDGE_K, EDGE_K, 1),
                         (EP // EDGE_K, EDGE_K, 16)),
        ((0, 2), (0, 0), (0, 0)))
    iw = jnp.pad(idx_w, (0, NP - N)).reshape(NP, 1)
    xb = jnp.pad(x_batch, (0, NP - N), constant_values=NDOC).reshape(NP, 1)
    zeros_half = jnp.zeros((NP, 1, 128), f32)

    x = _make_sc_gather(50000, D_IN, NP, 320)(word_vec, idx_pad)
    agg = _make_sc_edge()(x.reshape(NP, 2, 128), epk, ew_rep,
                          zeros_half).reshape(NP, D_IN)

    h, stats = _tc_h()(
        agg, x, iw, W_rel, W_root, b_rel.reshape(1, NWID))
    enc1, doc = _tc_enc()(
        h, stats, x, xb, bn1_gamma.reshape(1, NWID), bn1_beta.reshape(1, NWID),
        W_fc1, b_fc1.reshape(1, ENC_NH), W_fc2, b_fc2.reshape(1, ENC_NH))
    mean, logvar, docproj = _tc_doc()(
        doc, W_mean, b_mean.reshape(1, NT), bn_mean_gamma.reshape(1, NT),
        bn_mean_beta.reshape(1, NT), W_logvar, b_logvar.reshape(1, NT),
        W_phi[NWID + D_IN:, :])
    phi = _tc_phi()(enc1, xb, docproj, W_phi[:NWID + D_IN, :],
                    b_phi.reshape(1, NT))
    return (mean, logvar, phi[:N])


# speedup vs baseline: 1.0508x; 1.0499x over previous
import functools

import jax
import jax.numpy as jnp
from jax import lax
from jax.experimental import pallas as pl
from jax.experimental.pallas import tpu as pltpu
import jax.experimental.pallas.tpu_sc as plsc

N = 10000
NP = 10240
E = 160000
EP = 163840
D_IN = 256
NWID = 512
ENC_NH = 512
NT = 128
NDOC = 64
EPS = 1e-5
NC = 2
NS = 16
EDGE_K = 80
BLK = 1024
GRID = NP // BLK

def _sc_mesh():
    return plsc.VectorSubcoreMesh(
        core_axis_name="c", subcore_axis_name="s",
        num_cores=NC, num_subcores=NS)


def _make_sc_gather(V, D, B, K):
    bpw = B // (NC * NS)
    nchunks = bpw // K

    def body(table_hbm, idx_hbm, out_hbm, idx_v, rows_v, sem):
        wid = lax.axis_index("s") * NC + lax.axis_index("c")
        base = wid * bpw

        def chunk(j, carry):
            off = base + j * K
            pltpu.sync_copy(idx_hbm.at[pl.ds(off, K)], idx_v)
            pltpu.async_copy(table_hbm.at[idx_v], rows_v, sem).wait()
            pltpu.sync_copy(rows_v, out_hbm.at[pl.ds(off, K)])
            return carry

        lax.fori_loop(0, nchunks, chunk, 0)

    return pl.kernel(
        body,
        out_type=jax.ShapeDtypeStruct((B, D), jnp.float32),
        scratch_types=[
            pltpu.VMEM((K,), jnp.int32),
            pltpu.VMEM((K, D), jnp.float32),
            pltpu.SemaphoreType.DMA,
        ],
        mesh=_sc_mesh(),
    )


def _make_sc_edge(K=EDGE_K):
    nchunks_total = EP // K
    ntile = nchunks_total // NS
    zrows = NP // NS

    def body(xv_hbm, epk_hbm, ewr_hbm, zeros_hbm, out_hbm,
             ebuf, ewb, rows, dstb, acc,
             isem0, isem1, gsem0, gsem1, ssem0, ssem1):
        c = lax.axis_index("c")
        s = lax.axis_index("s")
        isem = (isem0, isem1)
        gsem = (gsem0, gsem1)
        ssem = (ssem0, ssem1)
        pltpu.sync_copy(zeros_hbm.at[pl.ds(s * zrows, zrows)],
                        acc.at[pl.ds(s * zrows, zrows)])
        plsc.subcore_barrier()
        base = s * ntile

        def issue_idx(cid, sl):
            pltpu.async_copy(epk_hbm.at[cid], ebuf.at[sl], isem[sl])
            pltpu.async_copy(ewr_hbm.at[cid], ewb.at[sl], isem[sl])

        def wait_idx(sl):
            pltpu.make_async_copy(epk_hbm.at[0], ebuf.at[sl], isem[sl]).wait()
            pltpu.make_async_copy(ewr_hbm.at[0], ewb.at[sl], isem[sl]).wait()

        H = K // 2

        def issue_gather(sl):
            pltpu.async_copy(xv_hbm.at[ebuf.at[sl, 0, pl.ds(0, H)],
                                       pl.ds(c, 1), :],
                             rows.at[sl, pl.ds(0, H)], gsem[sl])
            pltpu.async_copy(xv_hbm.at[ebuf.at[sl, 0, pl.ds(H, H)],
                                       pl.ds(c, 1), :],
                             rows.at[sl, pl.ds(H, H)], gsem[sl])

        def wait_gather(sl):
            pltpu.make_async_copy(xv_hbm.at[ebuf.at[sl, 0, pl.ds(0, H)],
                                            pl.ds(c, 1), :],
                                  rows.at[sl, pl.ds(0, H)], gsem[sl]).wait()
            pltpu.make_async_copy(xv_hbm.at[ebuf.at[sl, 0, pl.ds(H, H)],
                                            pl.ds(c, 1), :],
                                  rows.at[sl, pl.ds(H, H)], gsem[sl]).wait()

        def issue_scatter(sl):
            pltpu.async_copy(rows.at[sl], acc.at[dstb.at[sl]], ssem[sl],
                             add=True)

        def wait_scatter(sl):
            pltpu.make_async_copy(rows.at[sl], acc.at[dstb.at[sl]],
                                  ssem[sl]).wait()

        def scale(sl):
            @plsc.parallel_loop(0, K, step=1, unroll=4)
            def _(e):
                ew_bc = ewb[sl, e, pl.ds(0, 16)]
                for jj in range(8):
                    cs = pl.ds(jj * 16, 16)
                    rows[sl, e, 0, cs] = rows[sl, e, 0, cs] * ew_bc

        def step(j, sl, first=False):
            o = 1 - sl
            if not first:
                wait_scatter(o)
            wait_idx(o)
            issue_gather(o)
            wait_gather(sl)
            scale(sl)
            for q in range(K // 16):
                qs = pl.ds(q * 16, 16)
                dstb[sl, qs] = ebuf[sl, 1, qs]
            issue_idx(base + j + 2, sl)
            issue_scatter(sl)

        issue_idx(base, 0)
        issue_idx(base + 1, 1)
        wait_idx(0)
        issue_gather(0)
        step(0, 0, first=True)

        def pair(jj, carry):
            j0 = 1 + 2 * jj
            step(j0, 1)
            step(j0 + 1, 0)
            return carry

        lax.fori_loop(0, (ntile - 2) // 2, pair, 0)
        step(ntile - 1, 1)
        wait_idx(1)
        wait_gather(0)
        wait_scatter(1)
        plsc.subcore_barrier()
        pltpu.sync_copy(acc.at[pl.ds(s * zrows, zrows)],
                        out_hbm.at[pl.ds(s * zrows, zrows), pl.ds(c, 1), :])

    return pl.kernel(
        body,
        out_type=jax.ShapeDtypeStruct((NP, 2, 128), jnp.float32),
        scratch_types=[
            pltpu.VMEM((2, 2, K), jnp.int32),
            pltpu.VMEM((2, K, 16), jnp.float32),
            pltpu.VMEM((2, K, 1, 128), jnp.float32),
            pltpu.VMEM((2, K), jnp.int32),
            pltpu.VMEM_SHARED((NP, 1, 128), jnp.float32),
            pltpu.SemaphoreType.DMA,
            pltpu.SemaphoreType.DMA,
            pltpu.SemaphoreType.DMA,
            pltpu.SemaphoreType.DMA,
            pltpu.SemaphoreType.DMA,
            pltpu.SemaphoreType.DMA,
        ],
        mesh=_sc_mesh(),
    )


def _bdot(a, b):
    return jnp.dot(a.astype(jnp.bfloat16), b.astype(jnp.bfloat16),
                   preferred_element_type=jnp.float32)


def _fused_body(agg_ref, x_ref, iw_ref, xb_ref,
                wrel_ref, wroot_ref, brel_ref, g1_ref, b1_ref,
                wf1t_ref, wf1x_ref, bf1_ref, wf2t_ref, wf2x_ref, bf2_ref,
                wm_ref, bm_ref, gm_ref, btm_ref, wl_ref, bl_ref,
                wpb_ref, wpat_ref, wpax_ref, bp_ref,
                mean_ref, logvar_ref, phi_ref,
                h_scr, st_scr, doc_scr, dph_scr):
    i = pl.program_id(0)
    x = x_ref[...]
    oh = (xb_ref[...] ==
          lax.broadcasted_iota(jnp.int32, (1, NDOC), 1)).astype(jnp.float32)

    @pl.when(i < GRID)
    def _phase0():
        agg = agg_ref[...] + iw_ref[...] * x
        h = (_bdot(agg, wrel_ref[...]) + _bdot(x, wroot_ref[...])
             + brel_ref[...])
        h_scr[pl.ds(i * BLK, BLK), :] = h
        gid = i * BLK + lax.broadcasted_iota(jnp.int32, (BLK, 1), 0)
        hm = h * (gid < N).astype(jnp.float32)
        blkstats = jnp.concatenate(
            [jnp.sum(hm, axis=0, keepdims=True),
             jnp.sum(h * hm, axis=0, keepdims=True),
             jnp.zeros((6, NWID), jnp.float32)], axis=0)

        @pl.when(i == 0)
        def _():
            st_scr[...] = jnp.zeros_like(st_scr)

        st_scr[...] += blkstats

    def bn_tanh(r):
        m = st_scr[0, :] * (1.0 / N)
        v = st_scr[1, :] * (1.0 / N) - m * m
        inv = lax.rsqrt(v + EPS)
        h = h_scr[pl.ds(r * BLK, BLK), :]
        return jnp.tanh((h - m[None, :]) * inv[None, :] * g1_ref[...]
                        + b1_ref[...])

    @pl.when((i >= GRID) & (i < 2 * GRID))
    def _phase1():
        t = bn_tanh(i - GRID)
        a = jax.nn.sigmoid(_bdot(t, wf1t_ref[...]) + _bdot(x, wf1x_ref[...])
                           + bf1_ref[...])
        b = jnp.tanh(_bdot(t, wf2t_ref[...]) + _bdot(x, wf2x_ref[...])
                     + bf2_ref[...])
        part = lax.dot_general(oh, a * b, (((0,), (0,)), ((), ())),
                               preferred_element_type=jnp.float32,
                               precision=lax.Precision.HIGHEST)

        @pl.when(i == GRID)
        def _():
            doc_scr[...] = jnp.zeros_like(doc_scr)

        doc_scr[...] += part

    @pl.when(i >= 2 * GRID)
    def _phase2():
        @pl.when(i == 2 * GRID)
        def _doc_head():
            d = doc_scr[...]
            mp = _bdot(d, wm_ref[...]) + bm_ref[...]
            mm = jnp.mean(mp, axis=0, keepdims=True)
            vv = jnp.mean(mp * mp, axis=0, keepdims=True) - mm * mm
            mean_ref[...] = ((mp - mm) * lax.rsqrt(vv + EPS) * gm_ref[...]
                             + btm_ref[...])
            logvar_ref[...] = _bdot(d, wl_ref[...]) + bl_ref[...]
            dph_scr[...] = _bdot(d, wpb_ref[...])

        t = bn_tanh(i - 2 * GRID)
        logits = (_bdot(t, wpat_ref[...]) + _bdot(x, wpax_ref[...])
                  + jnp.dot(oh, dph_scr[...],
                            preferred_element_type=jnp.float32,
                            precision=lax.Precision.HIGHEST)
                  + bp_ref[...])
        z = logits - jnp.max(logits, axis=1, keepdims=True)
        ez = jnp.exp(z)
        phi_ref[...] = ez / jnp.sum(ez, axis=1, keepdims=True)


def _tc_fused():
    full = lambda i: (0, 0)
    rows = lambda i: (i % GRID, 0)
    p0rows = lambda i: (jnp.minimum(i, GRID - 1), 0)
    return pl.pallas_call(
        _fused_body,
        grid=(3 * GRID,),
        in_specs=[
            pl.BlockSpec((BLK, D_IN), p0rows),
            pl.BlockSpec((BLK, D_IN), rows),
            pl.BlockSpec((BLK, 1), p0rows),
            pl.BlockSpec((BLK, 1), rows),
            pl.BlockSpec((D_IN, NWID), full),
            pl.BlockSpec((D_IN, NWID), full),
            pl.BlockSpec((1, NWID), full),
            pl.BlockSpec((1, NWID), full),
            pl.BlockSpec((1, NWID), full),
            pl.BlockSpec((NWID, ENC_NH), full),
            pl.BlockSpec((D_IN, ENC_NH), full),
            pl.BlockSpec((1, ENC_NH), full),
            pl.BlockSpec((NWID, ENC_NH), full),
            pl.BlockSpec((D_IN, ENC_NH), full),
            pl.BlockSpec((1, ENC_NH), full),
            pl.BlockSpec((ENC_NH, NT), full),
            pl.BlockSpec((1, NT), full),
            pl.BlockSpec((1, NT), full),
            pl.BlockSpec((1, NT), full),
            pl.BlockSpec((ENC_NH, NT), full),
            pl.BlockSpec((1, NT), full),
            pl.BlockSpec((ENC_NH, NT), full),
            pl.BlockSpec((NWID, NT), full),
            pl.BlockSpec((D_IN, NT), full),
            pl.BlockSpec((1, NT), full),
        ],
        out_specs=[
            pl.BlockSpec((NDOC, NT), full),
            pl.BlockSpec((NDOC, NT), full),
            pl.BlockSpec((BLK, NT), lambda i: (jnp.maximum(i - 2 * GRID, 0), 0)),
        ],
        out_shape=[
            jax.ShapeDtypeStruct((NDOC, NT), jnp.float32),
            jax.ShapeDtypeStruct((NDOC, NT), jnp.float32),
            jax.ShapeDtypeStruct((NP, NT), jnp.float32),
        ],
        scratch_shapes=[
            pltpu.VMEM((NP, NWID), jnp.float32),
            pltpu.VMEM((8, NWID), jnp.float32),
            pltpu.VMEM((NDOC, ENC_NH), jnp.float32),
            pltpu.VMEM((NDOC, NT), jnp.float32),
        ],
    )


def kernel(idx_x, idx_w, x_batch, edge_index, edge_weight, word_vec,
           W_rel, b_rel, W_root, bn1_gamma, bn1_beta,
           W_fc1, b_fc1, W_fc2, b_fc2,
           W_mean, b_mean, bn_mean_gamma, bn_mean_beta,
           W_logvar, b_logvar, W_phi, b_phi):
    f32 = jnp.float32
    idx_pad = jnp.pad(idx_x.astype(jnp.int32), (0, NP - N))
    src = jnp.pad(edge_index[0].astype(jnp.int32), (0, EP - E))
    dst = jnp.pad(edge_index[1].astype(jnp.int32), (0, EP - E))
    ew_pad = jnp.pad(edge_weight, (0, EP - E))
    epk = jnp.pad(jnp.stack([src.reshape(-1, EDGE_K), dst.reshape(-1, EDGE_K)],
                            axis=1),
                  ((0, 2), (0, 0), (0, 0)))
    ew_rep = jnp.pad(
        jnp.broadcast_to(ew_pad.reshape(EP // EDGE_K, EDGE_K, 1),
                         (EP // EDGE_K, EDGE_K, 16)),
        ((0, 2), (0, 0), (0, 0)))
    iw = jnp.pad(idx_w, (0, NP - N)).reshape(NP, 1)
    xb = jnp.pad(x_batch, (0, NP - N), constant_values=NDOC).reshape(NP, 1)
    zeros_half = jnp.zeros((NP, 1, 128), f32)

    x = _make_sc_gather(50000, D_IN, NP, 320)(word_vec, idx_pad)
    agg = _make_sc_edge()(x.reshape(NP, 2, 128), epk, ew_rep,
                          zeros_half).reshape(NP, D_IN)

    mean, logvar, phi = _tc_fused()(
        agg, x, iw, xb,
        W_rel, W_root, b_rel.reshape(1, NWID),
        bn1_gamma.reshape(1, NWID), bn1_beta.reshape(1, NWID),
        W_fc1[:NWID], W_fc1[NWID:], b_fc1.reshape(1, ENC_NH),
        W_fc2[:NWID], W_fc2[NWID:], b_fc2.reshape(1, ENC_NH),
        W_mean, b_mean.reshape(1, NT), bn_mean_gamma.reshape(1, NT),
        bn_mean_beta.reshape(1, NT), W_logvar, b_logvar.reshape(1, NT),
        W_phi[NWID + D_IN:], W_phi[:NWID], W_phi[NWID:NWID + D_IN],
        b_phi.reshape(1, NT))
    return (mean, logvar, phi[:N])


# speedup vs baseline: 1.0523x; 1.0014x over previous
import functools

import jax
import jax.numpy as jnp
from jax import lax
from jax.experimental import pallas as pl
from jax.experimental.pallas import tpu as pltpu
import jax.experimental.pallas.tpu_sc as plsc

N = 10000
NP = 10240
E = 160000
EP = 163840
D_IN = 256
NWID = 512
ENC_NH = 512
NT = 128
NDOC = 64
EPS = 1e-5
NC = 2
NS = 16
EDGE_K = 80
BLK = 1024
GRID = NP // BLK

def _sc_mesh():
    return plsc.VectorSubcoreMesh(
        core_axis_name="c", subcore_axis_name="s",
        num_cores=NC, num_subcores=NS)


def _make_sc_gather(V, D, B, K):
    bpw = B // (NC * NS)
    nchunks = bpw // K

    def body(table_hbm, idx_hbm, out_hbm, idx_v, rows_v, sem):
        wid = lax.axis_index("s") * NC + lax.axis_index("c")
        base = wid * bpw

        def chunk(j, carry):
            off = base + j * K
            pltpu.sync_copy(idx_hbm.at[pl.ds(off, K)], idx_v)
            pltpu.async_copy(table_hbm.at[idx_v], rows_v, sem).wait()
            pltpu.sync_copy(rows_v, out_hbm.at[pl.ds(off, K)])
            return carry

        lax.fori_loop(0, nchunks, chunk, 0)

    return pl.kernel(
        body,
        out_type=jax.ShapeDtypeStruct((B, D), jnp.float32),
        scratch_types=[
            pltpu.VMEM((K,), jnp.int32),
            pltpu.VMEM((K, D), jnp.float32),
            pltpu.SemaphoreType.DMA,
        ],
        mesh=_sc_mesh(),
    )


def _make_sc_edge(K=EDGE_K):
    nchunks_total = EP // K
    ntile = nchunks_total // NS
    zrows = NP // NS

    def body(xv_hbm, epk_hbm, ewr_hbm, zeros_hbm, out_hbm,
             ebuf, ewb, rows, dstb, acc,
             isem0, isem1, gsem0, gsem1, ssem0, ssem1):
        c = lax.axis_index("c")
        s = lax.axis_index("s")
        isem = (isem0, isem1)
        gsem = (gsem0, gsem1)
        ssem = (ssem0, ssem1)
        pltpu.sync_copy(zeros_hbm.at[pl.ds(s * zrows, zrows)],
                        acc.at[pl.ds(s * zrows, zrows)])
        plsc.subcore_barrier()
        base = s * ntile

        def issue_idx(cid, sl):
            pltpu.async_copy(epk_hbm.at[cid], ebuf.at[sl], isem[sl])
            pltpu.async_copy(ewr_hbm.at[cid], ewb.at[sl], isem[sl])

        def wait_idx(sl):
            pltpu.make_async_copy(epk_hbm.at[0], ebuf.at[sl], isem[sl]).wait()
            pltpu.make_async_copy(ewr_hbm.at[0], ewb.at[sl], isem[sl]).wait()

        H = K // 2

        def issue_gather(sl):
            pltpu.async_copy(xv_hbm.at[ebuf.at[sl, 0, pl.ds(0, H)],
                                       pl.ds(c, 1), :],
                             rows.at[sl, pl.ds(0, H)], gsem[sl])
            pltpu.async_copy(xv_hbm.at[ebuf.at[sl, 0, pl.ds(H, H)],
                                       pl.ds(c, 1), :],
                             rows.at[sl, pl.ds(H, H)], gsem[sl])

        def wait_gather(sl):
            pltpu.make_async_copy(xv_hbm.at[ebuf.at[sl, 0, pl.ds(0, H)],
                                            pl.ds(c, 1), :],
                                  rows.at[sl, pl.ds(0, H)], gsem[sl]).wait()
            pltpu.make_async_copy(xv_hbm.at[ebuf.at[sl, 0, pl.ds(H, H)],
                                            pl.ds(c, 1), :],
                                  rows.at[sl, pl.ds(H, H)], gsem[sl]).wait()

        def issue_scatter(sl):
            pltpu.async_copy(rows.at[sl], acc.at[dstb.at[sl]], ssem[sl],
                             add=True)

        def wait_scatter(sl):
            pltpu.make_async_copy(rows.at[sl], acc.at[dstb.at[sl]],
                                  ssem[sl]).wait()

        def scale(sl):
            @plsc.parallel_loop(0, K, step=1, unroll=4)
            def _(e):
                ew_bc = ewb[sl, e, pl.ds(0, 16)]
                for jj in range(8):
                    cs = pl.ds(jj * 16, 16)
                    rows[sl, e, 0, cs] = rows[sl, e, 0, cs] * ew_bc

        def step(j, sl, first=False):
            o = 1 - sl
            if not first:
                wait_scatter(o)
            wait_idx(o)
            issue_gather(o)
            wait_gather(sl)
            scale(sl)
            for q in range(K // 16):
                qs = pl.ds(q * 16, 16)
                dstb[sl, qs] = ebuf[sl, 1, qs]
            issue_idx(base + j + 2, sl)
            issue_scatter(sl)

        issue_idx(base, 0)
        issue_idx(base + 1, 1)
        wait_idx(0)
        issue_gather(0)
        step(0, 0, first=True)

        def pair(jj, carry):
            j0 = 1 + 2 * jj
            step(j0, 1)
            step(j0 + 1, 0)
            return carry

        lax.fori_loop(0, (ntile - 2) // 2, pair, 0)
        step(ntile - 1, 1)
        wait_idx(1)
        wait_gather(0)
        wait_scatter(1)
        plsc.subcore_barrier()
        pltpu.sync_copy(acc.at[pl.ds(s * zrows, zrows)],
                        out_hbm.at[pl.ds(s * zrows, zrows), pl.ds(c, 1), :])

    return pl.kernel(
        body,
        out_type=jax.ShapeDtypeStruct((NP, 2, 128), jnp.float32),
        scratch_types=[
            pltpu.VMEM((2, 2, K), jnp.int32),
            pltpu.VMEM((2, K, 16), jnp.float32),
            pltpu.VMEM((2, K, 1, 128), jnp.float32),
            pltpu.VMEM((2, K), jnp.int32),
            pltpu.VMEM_SHARED((NP, 1, 128), jnp.float32),
            pltpu.SemaphoreType.DMA,
            pltpu.SemaphoreType.DMA,
            pltpu.SemaphoreType.DMA,
            pltpu.SemaphoreType.DMA,
            pltpu.SemaphoreType.DMA,
            pltpu.SemaphoreType.DMA,
        ],
        mesh=_sc_mesh(),
    )


def _bdot(a, b):
    return jnp.dot(a.astype(jnp.bfloat16), b.astype(jnp.bfloat16),
                   preferred_element_type=jnp.float32)


def _fused_body(agg_ref, x_ref, iw_ref, xb_ref,
                wrel_ref, wroot_ref, brel_ref, g1_ref, b1_ref,
                wf1t_ref, wf1x_ref, bf1_ref, wf2t_ref, wf2x_ref, bf2_ref,
                wm_ref, bm_ref, gm_ref, btm_ref, wl_ref, bl_ref,
                wpb_ref, wpat_ref, wpax_ref, bp_ref,
                mean_ref, logvar_ref, phi_ref,
                h_scr, st_scr, doc_scr, dph_scr, x_scr):
    i = pl.program_id(0)
    r = i % GRID
    x = x_ref[...]
    oh = (xb_ref[...] ==
          lax.broadcasted_iota(jnp.int32, (1, NDOC), 1)).astype(jnp.float32)

    @pl.when(i < GRID)
    def _phase0():
        x_scr[pl.ds(i * BLK, BLK), :] = x
        agg = agg_ref[...] + iw_ref[...] * x
        h = (_bdot(agg, wrel_ref[...]) + _bdot(x, wroot_ref[...])
             + brel_ref[...])
        h_scr[pl.ds(i * BLK, BLK), :] = h
        gid = i * BLK + lax.broadcasted_iota(jnp.int32, (BLK, 1), 0)
        hm = h * (gid < N).astype(jnp.float32)
        blkstats = jnp.concatenate(
            [jnp.sum(hm, axis=0, keepdims=True),
             jnp.sum(h * hm, axis=0, keepdims=True),
             jnp.zeros((6, NWID), jnp.float32)], axis=0)

        @pl.when(i == 0)
        def _():
            st_scr[...] = jnp.zeros_like(st_scr)

        st_scr[...] += blkstats

    def bn_tanh(r):
        m = st_scr[0, :] * (1.0 / N)
        v = st_scr[1, :] * (1.0 / N) - m * m
        inv = lax.rsqrt(v + EPS)
        h = h_scr[pl.ds(r * BLK, BLK), :]
        return jnp.tanh((h - m[None, :]) * inv[None, :] * g1_ref[...]
                        + b1_ref[...])

    @pl.when((i >= GRID) & (i < 2 * GRID))
    def _phase1():
        x = x_scr[pl.ds(r * BLK, BLK), :]
        t = bn_tanh(i - GRID)
        a = jax.nn.sigmoid(_bdot(t, wf1t_ref[...]) + _bdot(x, wf1x_ref[...])
                           + bf1_ref[...])
        b = jnp.tanh(_bdot(t, wf2t_ref[...]) + _bdot(x, wf2x_ref[...])
                     + bf2_ref[...])
        part = lax.dot_general(oh, a * b, (((0,), (0,)), ((), ())),
                               preferred_element_type=jnp.float32,
                               precision=lax.Precision.HIGHEST)

        @pl.when(i == GRID)
        def _():
            doc_scr[...] = jnp.zeros_like(doc_scr)

        doc_scr[...] += part

    @pl.when(i >= 2 * GRID)
    def _phase2():
        @pl.when(i == 2 * GRID)
        def _doc_head():
            d = doc_scr[...]
            mp = _bdot(d, wm_ref[...]) + bm_ref[...]
            mm = jnp.mean(mp, axis=0, keepdims=True)
            vv = jnp.mean(mp * mp, axis=0, keepdims=True) - mm * mm
            mean_ref[...] = ((mp - mm) * lax.rsqrt(vv + EPS) * gm_ref[...]
                             + btm_ref[...])
            logvar_ref[...] = _bdot(d, wl_ref[...]) + bl_ref[...]
            dph_scr[...] = _bdot(d, wpb_ref[...])

        x = x_scr[pl.ds(r * BLK, BLK), :]
        t = bn_tanh(i - 2 * GRID)
        logits = (_bdot(t, wpat_ref[...]) + _bdot(x, wpax_ref[...])
                  + jnp.dot(oh, dph_scr[...],
                            preferred_element_type=jnp.float32,
                            precision=lax.Precision.HIGHEST)
                  + bp_ref[...])
        z = logits - jnp.max(logits, axis=1, keepdims=True)
        ez = jnp.exp(z)
        phi_ref[...] = ez / jnp.sum(ez, axis=1, keepdims=True)


def _tc_fused():
    full = lambda i: (0, 0)
    rows = lambda i: (i % GRID, 0)
    p0rows = lambda i: (jnp.minimum(i, GRID - 1), 0)
    return pl.pallas_call(
        _fused_body,
        grid=(3 * GRID,),
        in_specs=[
            pl.BlockSpec((BLK, D_IN), p0rows),
            pl.BlockSpec((BLK, D_IN), p0rows),
            pl.BlockSpec((BLK, 1), p0rows),
            pl.BlockSpec((BLK, 1), rows),
            pl.BlockSpec((D_IN, NWID), full),
            pl.BlockSpec((D_IN, NWID), full),
            pl.BlockSpec((1, NWID), full),
            pl.BlockSpec((1, NWID), full),
            pl.BlockSpec((1, NWID), full),
            pl.BlockSpec((NWID, ENC_NH), full),
            pl.BlockSpec((D_IN, ENC_NH), full),
            pl.BlockSpec((1, ENC_NH), full),
            pl.BlockSpec((NWID, ENC_NH), full),
            pl.BlockSpec((D_IN, ENC_NH), full),
            pl.BlockSpec((1, ENC_NH), full),
            pl.BlockSpec((ENC_NH, NT), full),
            pl.BlockSpec((1, NT), full),
            pl.BlockSpec((1, NT), full),
            pl.BlockSpec((1, NT), full),
            pl.BlockSpec((ENC_NH, NT), full),
            pl.BlockSpec((1, NT), full),
            pl.BlockSpec((ENC_NH, NT), full),
            pl.BlockSpec((NWID, NT), full),
            pl.BlockSpec((D_IN, NT), full),
            pl.BlockSpec((1, NT), full),
        ],
        out_specs=[
            pl.BlockSpec((NDOC, NT), full),
            pl.BlockSpec((NDOC, NT), full),
            pl.BlockSpec((BLK, NT), lambda i: (jnp.maximum(i - 2 * GRID, 0), 0)),
        ],
        out_shape=[
            jax.ShapeDtypeStruct((NDOC, NT), jnp.float32),
            jax.ShapeDtypeStruct((NDOC, NT), jnp.float32),
            jax.ShapeDtypeStruct((NP, NT), jnp.float32),
        ],
        scratch_shapes=[
            pltpu.VMEM((NP, NWID), jnp.float32),
            pltpu.VMEM((8, NWID), jnp.float32),
            pltpu.VMEM((NDOC, ENC_NH), jnp.float32),
            pltpu.VMEM((NDOC, NT), jnp.float32),
            pltpu.VMEM((NP, D_IN), jnp.float32),
        ],
    )


def kernel(idx_x, idx_w, x_batch, edge_index, edge_weight, word_vec,
           W_rel, b_rel, W_root, bn1_gamma, bn1_beta,
           W_fc1, b_fc1, W_fc2, b_fc2,
           W_mean, b_mean, bn_mean_gamma, bn_mean_beta,
           W_logvar, b_logvar, W_phi, b_phi):
    f32 = jnp.float32
    idx_pad = jnp.pad(idx_x.astype(jnp.int32), (0, NP - N))
    src = jnp.pad(edge_index[0].astype(jnp.int32), (0, EP - E))
    dst = jnp.pad(edge_index[1].astype(jnp.int32), (0, EP - E))
    ew_pad = jnp.pad(edge_weight, (0, EP - E))
    epk = jnp.pad(jnp.stack([src.reshape(-1, EDGE_K), dst.reshape(-1, EDGE_K)],
                            axis=1),
                  ((0, 2), (0, 0), (0, 0)))
    ew_rep = jnp.pad(
        jnp.broadcast_to(ew_pad.reshape(EP // EDGE_K, EDGE_K, 1),
                         (EP // EDGE_K, EDGE_K, 16)),
        ((0, 2), (0, 0), (0, 0)))
    iw = jnp.pad(idx_w, (0, NP - N)).reshape(NP, 1)
    xb = jnp.pad(x_batch, (0, NP - N), constant_values=NDOC).reshape(NP, 1)
    zeros_half = jnp.zeros((NP, 1, 128), f32)

    x = _make_sc_gather(50000, D_IN, NP, 320)(word_vec, idx_pad)
    agg = _make_sc_edge()(x.reshape(NP, 2, 128), epk, ew_rep,
                          zeros_half).reshape(NP, D_IN)

    mean, logvar, phi = _tc_fused()(
        agg, x, iw, xb,
        W_rel, W_root, b_rel.reshape(1, NWID),
        bn1_gamma.reshape(1, NWID), bn1_beta.reshape(1, NWID),
        W_fc1[:NWID].astype(jnp.bfloat16), W_fc1[NWID:].astype(jnp.bfloat16),
        b_fc1.reshape(1, ENC_NH),
        W_fc2[:NWID].astype(jnp.bfloat16), W_fc2[NWID:].astype(jnp.bfloat16),
        b_fc2.reshape(1, ENC_NH),
        W_mean, b_mean.reshape(1, NT), bn_mean_gamma.reshape(1, NT),
        bn_mean_beta.reshape(1, NT), W_logvar, b_logvar.reshape(1, NT),
        W_phi[NWID + D_IN:], W_phi[:NWID], W_phi[NWID:NWID + D_IN],
        b_phi.reshape(1, NT))
    return (mean, logvar, phi[:N])


# speedup vs baseline: 1.0533x; 1.0009x over previous
import jax
import jax.numpy as jnp
from jax import lax
from jax.experimental import pallas as pl
from jax.experimental.pallas import tpu as pltpu
import jax.experimental.pallas.tpu_sc as plsc

N = 10000
NP = 10240
E = 160000
EP = 163840
D_IN = 256
NWID = 512
ENC_NH = 512
NT = 128
NDOC = 64
EPS = 1e-5
NC = 2
NS = 16
EDGE_K = 80
BLK = 1024
GRID = NP // BLK

def _sc_mesh():
    return plsc.VectorSubcoreMesh(
        core_axis_name="c", subcore_axis_name="s",
        num_cores=NC, num_subcores=NS)


def _make_sc_gather(V, D, B, K):
    bpw = B // (NC * NS)
    nchunks = bpw // K

    def body(table_hbm, idx_hbm, out_hbm, idx_v, rows_v, sem):
        wid = lax.axis_index("s") * NC + lax.axis_index("c")
        base = wid * bpw

        def chunk(j, carry):
            off = base + j * K
            pltpu.sync_copy(idx_hbm.at[pl.ds(off, K)], idx_v)
            pltpu.async_copy(table_hbm.at[idx_v], rows_v, sem).wait()
            pltpu.sync_copy(rows_v, out_hbm.at[pl.ds(off, K)])
            return carry

        lax.fori_loop(0, nchunks, chunk, 0)

    return pl.kernel(
        body,
        out_type=jax.ShapeDtypeStruct((B, D), jnp.float32),
        scratch_types=[
            pltpu.VMEM((K,), jnp.int32),
            pltpu.VMEM((K, D), jnp.float32),
            pltpu.SemaphoreType.DMA,
        ],
        mesh=_sc_mesh(),
    )


def _make_sc_edge(K=EDGE_K):
    nchunks_total = EP // K
    ntile = nchunks_total // NS
    zrows = NP // NS

    def body(xv_hbm, epk_hbm, ewr_hbm, zeros_hbm, out_hbm,
             ebuf, ewb, rows, dstb, acc,
             isem0, isem1, gsem0, gsem1, ssem0, ssem1):
        c = lax.axis_index("c")
        s = lax.axis_index("s")
        isem = (isem0, isem1)
        gsem = (gsem0, gsem1)
        ssem = (ssem0, ssem1)
        pltpu.sync_copy(zeros_hbm.at[pl.ds(s * zrows, zrows)],
                        acc.at[pl.ds(s * zrows, zrows)])
        plsc.subcore_barrier()
        base = s * ntile

        def issue_idx(cid, sl):
            pltpu.async_copy(epk_hbm.at[cid], ebuf.at[sl], isem[sl])
            pltpu.async_copy(ewr_hbm.at[cid], ewb.at[sl], isem[sl])

        def wait_idx(sl):
            pltpu.make_async_copy(epk_hbm.at[0], ebuf.at[sl], isem[sl]).wait()
            pltpu.make_async_copy(ewr_hbm.at[0], ewb.at[sl], isem[sl]).wait()

        H = K // 2

        def issue_gather(sl):
            pltpu.async_copy(xv_hbm.at[ebuf.at[sl, 0, pl.ds(0, H)],
                                       pl.ds(c, 1), :],
                             rows.at[sl, pl.ds(0, H)], gsem[sl])
            pltpu.async_copy(xv_hbm.at[ebuf.at[sl, 0, pl.ds(H, H)],
                                       pl.ds(c, 1), :],
                             rows.at[sl, pl.ds(H, H)], gsem[sl])

        def wait_gather(sl):
            pltpu.make_async_copy(xv_hbm.at[ebuf.at[sl, 0, pl.ds(0, H)],
                                            pl.ds(c, 1), :],
                                  rows.at[sl, pl.ds(0, H)], gsem[sl]).wait()
            pltpu.make_async_copy(xv_hbm.at[ebuf.at[sl, 0, pl.ds(H, H)],
                                            pl.ds(c, 1), :],
                                  rows.at[sl, pl.ds(H, H)], gsem[sl]).wait()

        def issue_scatter(sl):
            pltpu.async_copy(rows.at[sl], acc.at[dstb.at[sl]], ssem[sl],
                             add=True)

        def wait_scatter(sl):
            pltpu.make_async_copy(rows.at[sl], acc.at[dstb.at[sl]],
                                  ssem[sl]).wait()

        def scale(sl):
            @plsc.parallel_loop(0, K, step=1, unroll=4)
            def _(e):
                ew_bc = ewb[sl, e, pl.ds(0, 16)]
                for jj in range(8):
                    cs = pl.ds(jj * 16, 16)
                    rows[sl, e, 0, cs] = rows[sl, e, 0, cs] * ew_bc

        def step(j, sl, first=False):
            o = 1 - sl
            if not first:
                wait_scatter(o)
            wait_idx(o)
            issue_gather(o)
            wait_gather(sl)
            scale(sl)
            for q in range(K // 16):
                qs = pl.ds(q * 16, 16)
                dstb[sl, qs] = ebuf[sl, 1, qs]
            issue_idx(base + j + 2, sl)
            issue_scatter(sl)

        issue_idx(base, 0)
        issue_idx(base + 1, 1)
        wait_idx(0)
        issue_gather(0)
        step(0, 0, first=True)

        def pair(jj, carry):
            j0 = 1 + 2 * jj
            step(j0, 1)
            step(j0 + 1, 0)
            return carry

        lax.fori_loop(0, (ntile - 2) // 2, pair, 0)
        step(ntile - 1, 1)
        wait_idx(1)
        wait_gather(0)
        wait_scatter(1)
        plsc.subcore_barrier()
        pltpu.sync_copy(acc.at[pl.ds(s * zrows, zrows)],
                        out_hbm.at[pl.ds(s * zrows, zrows), pl.ds(c, 1), :])

    return pl.kernel(
        body,
        out_type=jax.ShapeDtypeStruct((NP, 2, 128), jnp.float32),
        scratch_types=[
            pltpu.VMEM((2, 2, K), jnp.int32),
            pltpu.VMEM((2, K, 16), jnp.float32),
            pltpu.VMEM((2, K, 1, 128), jnp.float32),
            pltpu.VMEM((2, K), jnp.int32),
            pltpu.VMEM_SHARED((NP, 1, 128), jnp.float32),
            pltpu.SemaphoreType.DMA,
            pltpu.SemaphoreType.DMA,
            pltpu.SemaphoreType.DMA,
            pltpu.SemaphoreType.DMA,
            pltpu.SemaphoreType.DMA,
            pltpu.SemaphoreType.DMA,
        ],
        mesh=_sc_mesh(),
    )


def _bdot(a, b):
    return jnp.dot(a.astype(jnp.bfloat16), b.astype(jnp.bfloat16),
                   preferred_element_type=jnp.float32)


def _fused_body(agg_ref, x_ref, iw_ref, xb_ref,
                wrel_ref, wroot_ref, brel_ref, g1_ref, b1_ref,
                wf1t_ref, wf1x_ref, bf1_ref, wf2t_ref, wf2x_ref, bf2_ref,
                wm_ref, bm_ref, gm_ref, btm_ref, wl_ref, bl_ref,
                wpb_ref, wpat_ref, wpax_ref, bp_ref,
                mean_ref, logvar_ref, phi_ref,
                h_scr, st_scr, doc_scr, dph_scr, x_scr):
    i = pl.program_id(0)
    r = i % GRID
    x = x_ref[...]
    oh = (xb_ref[...] ==
          lax.broadcasted_iota(jnp.int32, (1, NDOC), 1)).astype(jnp.float32)

    @pl.when(i < GRID)
    def _phase0():
        x_scr[pl.ds(i * BLK, BLK), :] = x
        agg = agg_ref[...] + iw_ref[...] * x
        h = (_bdot(agg, wrel_ref[...]) + _bdot(x, wroot_ref[...])
             + brel_ref[...])
        h_scr[pl.ds(i * BLK, BLK), :] = h
        gid = i * BLK + lax.broadcasted_iota(jnp.int32, (BLK, 1), 0)
        hm = h * (gid < N).astype(jnp.float32)
        blkstats = jnp.concatenate(
            [jnp.sum(hm, axis=0, keepdims=True),
             jnp.sum(h * hm, axis=0, keepdims=True),
             jnp.zeros((6, NWID), jnp.float32)], axis=0)

        @pl.when(i == 0)
        def _():
            st_scr[...] = jnp.zeros_like(st_scr)

        st_scr[...] += blkstats

    def bn_tanh(r):
        m = st_scr[0, :] * (1.0 / N)
        v = st_scr[1, :] * (1.0 / N) - m * m
        inv = lax.rsqrt(v + EPS)
        h = h_scr[pl.ds(r * BLK, BLK), :]
        return jnp.tanh((h - m[None, :]) * inv[None, :] * g1_ref[...]
                        + b1_ref[...])

    @pl.when((i >= GRID) & (i < 2 * GRID))
    def _phase1():
        x = x_scr[pl.ds(r * BLK, BLK), :]
        t = bn_tanh(i - GRID)
        a = jax.nn.sigmoid(_bdot(t, wf1t_ref[...]) + _bdot(x, wf1x_ref[...])
                           + bf1_ref[...])
        b = jnp.tanh(_bdot(t, wf2t_ref[...]) + _bdot(x, wf2x_ref[...])
                     + bf2_ref[...])
        part = lax.dot_general(oh, a * b, (((0,), (0,)), ((), ())),
                               preferred_element_type=jnp.float32,
                               precision=lax.Precision.HIGHEST)

        @pl.when(i == GRID)
        def _():
            doc_scr[...] = jnp.zeros_like(doc_scr)

        doc_scr[...] += part

    @pl.when(i >= 2 * GRID)
    def _phase2():
        @pl.when(i == 2 * GRID)
        def _doc_head():
            d = doc_scr[...]
            mp = _bdot(d, wm_ref[...]) + bm_ref[...]
            mm = jnp.mean(mp, axis=0, keepdims=True)
            vv = jnp.mean(mp * mp, axis=0, keepdims=True) - mm * mm
            mean_ref[...] = ((mp - mm) * lax.rsqrt(vv + EPS) * gm_ref[...]
                             + btm_ref[...])
            logvar_ref[...] = _bdot(d, wl_ref[...]) + bl_ref[...]
            dph_scr[...] = _bdot(d, wpb_ref[...])

        x = x_scr[pl.ds(r * BLK, BLK), :]
        t = bn_tanh(i - 2 * GRID)
        logits = (_bdot(t, wpat_ref[...]) + _bdot(x, wpax_ref[...])
                  + jnp.dot(oh, dph_scr[...],
                            preferred_element_type=jnp.float32,
                            precision=lax.Precision.HIGHEST)
                  + bp_ref[...])
        z = logits - jnp.max(logits, axis=1, keepdims=True)
        ez = jnp.exp(z)
        phi_ref[...] = ez / jnp.sum(ez, axis=1, keepdims=True)


def _tc_fused():
    full = lambda i: (0, 0)
    rows = lambda i: (i % GRID, 0)
    p0rows = lambda i: (jnp.minimum(i, GRID - 1), 0)
    return pl.pallas_call(
        _fused_body,
        grid=(3 * GRID,),
        in_specs=[
            pl.BlockSpec((BLK, D_IN), p0rows),
            pl.BlockSpec((BLK, D_IN), p0rows),
            pl.BlockSpec((BLK, 1), p0rows),
            pl.BlockSpec((BLK, 1), rows),
            pl.BlockSpec((D_IN, NWID), full),
            pl.BlockSpec((D_IN, NWID), full),
            pl.BlockSpec((1, NWID), full),
            pl.BlockSpec((1, NWID), full),
            pl.BlockSpec((1, NWID), full),
            pl.BlockSpec((NWID, ENC_NH), full),
            pl.BlockSpec((D_IN, ENC_NH), full),
            pl.BlockSpec((1, ENC_NH), full),
            pl.BlockSpec((NWID, ENC_NH), full),
            pl.BlockSpec((D_IN, ENC_NH), full),
            pl.BlockSpec((1, ENC_NH), full),
            pl.BlockSpec((ENC_NH, NT), full),
            pl.BlockSpec((1, NT), full),
            pl.BlockSpec((1, NT), full),
            pl.BlockSpec((1, NT), full),
            pl.BlockSpec((ENC_NH, NT), full),
            pl.BlockSpec((1, NT), full),
            pl.BlockSpec((ENC_NH, NT), full),
            pl.BlockSpec((NWID, NT), full),
            pl.BlockSpec((D_IN, NT), full),
            pl.BlockSpec((1, NT), full),
        ],
        out_specs=[
            pl.BlockSpec((NDOC, NT), full),
            pl.BlockSpec((NDOC, NT), full),
            pl.BlockSpec((BLK, NT), lambda i: (jnp.maximum(i - 2 * GRID, 0), 0)),
        ],
        out_shape=[
            jax.ShapeDtypeStruct((NDOC, NT), jnp.float32),
            jax.ShapeDtypeStruct((NDOC, NT), jnp.float32),
            jax.ShapeDtypeStruct((NP, NT), jnp.float32),
        ],
        scratch_shapes=[
            pltpu.VMEM((NP, NWID), jnp.float32),
            pltpu.VMEM((8, NWID), jnp.float32),
            pltpu.VMEM((NDOC, ENC_NH), jnp.float32),
            pltpu.VMEM((NDOC, NT), jnp.float32),
            pltpu.VMEM((NP, D_IN), jnp.float32),
        ],
    )


def kernel(idx_x, idx_w, x_batch, edge_index, edge_weight, word_vec,
           W_rel, b_rel, W_root, bn1_gamma, bn1_beta,
           W_fc1, b_fc1, W_fc2, b_fc2,
           W_mean, b_mean, bn_mean_gamma, bn_mean_beta,
           W_logvar, b_logvar, W_phi, b_phi):
    f32 = jnp.float32
    idx_pad = jnp.pad(idx_x.astype(jnp.int32), (0, NP - N))
    src = jnp.pad(edge_index[0].astype(jnp.int32), (0, EP - E))
    dst = jnp.pad(edge_index[1].astype(jnp.int32), (0, EP - E))
    ew_pad = jnp.pad(edge_weight, (0, EP - E))
    epk = jnp.pad(jnp.stack([src.reshape(-1, EDGE_K), dst.reshape(-1, EDGE_K)],
                            axis=1),
                  ((0, 2), (0, 0), (0, 0)))
    ew_rep = jnp.pad(
        jnp.broadcast_to(ew_pad.reshape(EP // EDGE_K, EDGE_K, 1),
                         (EP // EDGE_K, EDGE_K, 16)),
        ((0, 2), (0, 0), (0, 0)))
    iw = jnp.pad(idx_w, (0, NP - N)).reshape(NP, 1)
    xb = jnp.pad(x_batch, (0, NP - N), constant_values=NDOC).reshape(NP, 1)
    zeros_half = jnp.zeros((NP, 1, 128), f32)

    x = _make_sc_gather(50000, D_IN, NP, 320)(word_vec, idx_pad)
    agg = _make_sc_edge()(x.reshape(NP, 2, 128), epk, ew_rep,
                          zeros_half).reshape(NP, D_IN)

    mean, logvar, phi = _tc_fused()(
        agg, x, iw, xb,
        W_rel, W_root, b_rel.reshape(1, NWID),
        bn1_gamma.reshape(1, NWID), bn1_beta.reshape(1, NWID),
        W_fc1[:NWID].astype(jnp.bfloat16), W_fc1[NWID:].astype(jnp.bfloat16),
        b_fc1.reshape(1, ENC_NH),
        W_fc2[:NWID].astype(jnp.bfloat16), W_fc2[NWID:].astype(jnp.bfloat16),
        b_fc2.reshape(1, ENC_NH),
        W_mean, b_mean.reshape(1, NT), bn_mean_gamma.reshape(1, NT),
        bn_mean_beta.reshape(1, NT), W_logvar, b_logvar.reshape(1, NT),
        W_phi[NWID + D_IN:], W_phi[:NWID], W_phi[NWID:NWID + D_IN],
        b_phi.reshape(1, NT))
    return (mean, logvar, phi[:N])
